# baseline (speedup 1.0000x reference)
import jax
import jax.numpy as jnp
from jax.experimental import pallas as pl
from jax.experimental.pallas import tpu as pltpu


def _dma_copy_body(x_hbm, o_hbm, sem):
    copy = pltpu.make_async_copy(x_hbm, o_hbm, sem)
    copy.start()
    copy.wait()


def _pallas_copy(joints):
    B, J, C = joints.shape
    flat = joints.reshape(B, J * C)
    out = pl.pallas_call(
        _dma_copy_body,
        in_specs=[pl.BlockSpec(memory_space=pltpu.MemorySpace.HBM)],
        out_specs=pl.BlockSpec(memory_space=pltpu.MemorySpace.HBM),
        scratch_shapes=[pltpu.SemaphoreType.DMA],
        out_shape=jax.ShapeDtypeStruct((B, J * C), flat.dtype),
    )(flat)
    return out.reshape(B, J, C)


def _copy_body(x_ref, o_ref):
    o_ref[...] = x_ref[...]


def _gather_concat_body(idx_ref, verts_ref, joints_ref, o_ref):
    J = joints_ref.shape[1]
    K = idx_ref.shape[0]
    o_ref[0, :J, :] = joints_ref[0, :, :]
    for k in range(K):
        o_ref[0, J + k, :] = verts_ref[0, idx_ref[k], :]


def kernel(vertices, joints, extra_joints_idxs):
    K = extra_joints_idxs.shape[0]
    if K == 0:
        return _pallas_copy(joints)

    B, J, C = joints.shape
    V = vertices.shape[1]
    return pl.pallas_call(
        _gather_concat_body,
        grid_spec=pltpu.PrefetchScalarGridSpec(
            num_scalar_prefetch=1,
            grid=(B,),
            in_specs=[
                pl.BlockSpec((1, V, C), lambda b, idx: (b, 0, 0)),
                pl.BlockSpec((1, J, C), lambda b, idx: (b, 0, 0)),
            ],
            out_specs=pl.BlockSpec((1, J + K, C), lambda b, idx: (b, 0, 0)),
        ),
        out_shape=jax.ShapeDtypeStruct((B, J + K, C), joints.dtype),
    )(extra_joints_idxs, vertices, joints)


# speedup vs baseline: 4.6025x; 4.6025x over previous
import jax
import jax.numpy as jnp
from jax.experimental import pallas as pl
from jax.experimental.pallas import tpu as pltpu


_NCHUNK = 8


def _dma_copy_body(x_hbm, o_hbm, vmem, sems):
    rows = x_hbm.shape[0] // _NCHUNK
    ins = [
        pltpu.make_async_copy(
            x_hbm.at[pl.ds(i * rows, rows)],
            vmem.at[pl.ds(i * rows, rows)],
            sems.at[i],
        )
        for i in range(_NCHUNK)
    ]
    for c in ins:
        c.start()
    outs = []
    for i in range(_NCHUNK):
        ins[i].wait()
        c = pltpu.make_async_copy(
            vmem.at[pl.ds(i * rows, rows)],
            o_hbm.at[pl.ds(i * rows, rows)],
            sems.at[i],
        )
        c.start()
        outs.append(c)
    for c in outs:
        c.wait()


def _pallas_copy(joints):
    B, J, C = joints.shape
    flat = joints.reshape(B, J * C)
    out = pl.pallas_call(
        _dma_copy_body,
        in_specs=[pl.BlockSpec(memory_space=pltpu.MemorySpace.HBM)],
        out_specs=pl.BlockSpec(memory_space=pltpu.MemorySpace.HBM),
        scratch_shapes=[
            pltpu.VMEM((B, J * C), flat.dtype),
            pltpu.SemaphoreType.DMA((_NCHUNK,)),
        ],
        out_shape=jax.ShapeDtypeStruct((B, J * C), flat.dtype),
    )(flat)
    return out.reshape(B, J, C)


def _copy_body(x_ref, o_ref):
    o_ref[...] = x_ref[...]


def _gather_concat_body(idx_ref, verts_ref, joints_ref, o_ref):
    J = joints_ref.shape[1]
    K = idx_ref.shape[0]
    o_ref[0, :J, :] = joints_ref[0, :, :]
    for k in range(K):
        o_ref[0, J + k, :] = verts_ref[0, idx_ref[k], :]


def kernel(vertices, joints, extra_joints_idxs):
    K = extra_joints_idxs.shape[0]
    if K == 0:
        return _pallas_copy(joints)

    B, J, C = joints.shape
    V = vertices.shape[1]
    return pl.pallas_call(
        _gather_concat_body,
        grid_spec=pltpu.PrefetchScalarGridSpec(
            num_scalar_prefetch=1,
            grid=(B,),
            in_specs=[
                pl.BlockSpec((1, V, C), lambda b, idx: (b, 0, 0)),
                pl.BlockSpec((1, J, C), lambda b, idx: (b, 0, 0)),
            ],
            out_specs=pl.BlockSpec((1, J + K, C), lambda b, idx: (b, 0, 0)),
        ),
        out_shape=jax.ShapeDtypeStruct((B, J + K, C), joints.dtype),
    )(extra_joints_idxs, vertices, joints)
